# baseline (speedup 1.0000x reference)
import functools

import jax
import jax.numpy as jnp
from jax import lax
from jax.experimental import pallas as pl
from jax.experimental.pallas import tpu as pltpu
from jax.experimental.pallas import tpu_sc as plsc

B = 16384
D = 128
H = 1024

CHUNKS = (4096, 12288)
MLP_BB = (4096, 4096)

_INFO = plsc.get_sparse_core_info()
_NC, _NS = _INFO.num_cores, _INFO.num_subcores
_NW = _NC * _NS
_CH = 128

_sc_mesh = plsc.VectorSubcoreMesh(core_axis_name="c", subcore_axis_name="s")


def _make_sc_gather(start, rows):
    bpw = rows // _NW
    ng = bpw // _CH

    @functools.partial(
        pl.kernel,
        mesh=_sc_mesh,
        out_type=[
            jax.ShapeDtypeStruct((rows, D), jnp.float32),
            jax.ShapeDtypeStruct((rows, D), jnp.float32),
        ],
        scratch_types=[
            pltpu.VMEM((bpw,), jnp.int32),
            pltpu.VMEM((bpw,), jnp.int32),
            pltpu.VMEM((bpw, D), jnp.float32),
            pltpu.VMEM((bpw, D), jnp.float32),
            pltpu.SemaphoreType.DMA,
            pltpu.SemaphoreType.DMA,
            pltpu.SemaphoreType.DMA,
        ],
    )
    def sc_gather(xu_hbm, xm_hbm, ut_hbm, mt_hbm, u_out, m_out,
                  idxu_v, idxm_v, urows_v, mrows_v, sem_u, sem_m, sem_w):
        wid = lax.axis_index("s") * _NC + lax.axis_index("c")
        base = wid * bpw

        pltpu.sync_copy(xu_hbm.at[pl.ds(base, bpw)], idxu_v)
        pltpu.sync_copy(xm_hbm.at[pl.ds(base, bpw)], idxm_v)
        gu = [
            pltpu.async_copy(
                ut_hbm.at[idxu_v.at[pl.ds(j * _CH, _CH)]],
                urows_v.at[pl.ds(j * _CH, _CH)], sem_u,
            )
            for j in range(ng)
        ]
        gm = [
            pltpu.async_copy(
                mt_hbm.at[idxm_v.at[pl.ds(j * _CH, _CH)]],
                mrows_v.at[pl.ds(j * _CH, _CH)], sem_m,
            )
            for j in range(ng)
        ]
        for cp in gu:
            cp.wait()
        wu = pltpu.async_copy(urows_v, u_out.at[pl.ds(base, bpw)], sem_w)
        for cp in gm:
            cp.wait()
        wm = pltpu.async_copy(mrows_v, m_out.at[pl.ds(base, bpw)], sem_w)
        wu.wait()
        wm.wait()

    return sc_gather


def _make_mlp_body(bb):
    def _mlp_body(u_ref, m_ref, w1t_ref, b1_ref, w2_ref, b2_ref, out_ref,
                  acc_ref=None):
        del acc_ref
        ut = u_ref[...].astype(jnp.bfloat16).T
        mt = m_ref[...].astype(jnp.bfloat16).T
        w1t = w1t_ref[...]
        hT = (
            jnp.dot(w1t[:, :D], ut, preferred_element_type=jnp.float32)
            + jnp.dot(w1t[:, D:], mt, preferred_element_type=jnp.float32)
            + b1_ref[...]
        )
        hT = jnp.maximum(hT, 0.0)
        g = jnp.dot(w2_ref[...], hT, preferred_element_type=jnp.float32)
        out_ref[...] = g.reshape(bb // 128, 128) + b2_ref[0, 0]
    return _mlp_body


def _mlp_chunk(start, rows, bb, u, m, w1t, b1r, w2r, b2r, acc=None):
    bbase = start // bb
    body = _make_mlp_body(bb)
    wrapped = body if acc is None else (
        lambda u_r, m_r, w_r, b1_r, w2_r, b2_r, a_r, o_r:
            body(u_r, m_r, w_r, b1_r, w2_r, b2_r, o_r, a_r)
    )
    in_specs = [
        pl.BlockSpec((bb, D), lambda i: (i, 0)),
        pl.BlockSpec((bb, D), lambda i: (i, 0)),
        pl.BlockSpec((H, 2 * D), lambda i: (0, 0)),
        pl.BlockSpec((H, 1), lambda i: (0, 0)),
        pl.BlockSpec((1, H), lambda i: (0, 0)),
        pl.BlockSpec((1, 1), lambda i: (0, 0)),
    ]
    args = [u, m, w1t, b1r, w2r, b2r]
    aliases = {}
    if acc is not None:
        in_specs.append(pl.BlockSpec(memory_space=pl.ANY))
        args.append(acc)
        aliases = {6: 0}
    return pl.pallas_call(
        wrapped,
        grid=(rows // bb,),
        in_specs=in_specs,
        out_specs=pl.BlockSpec(
            (bb // 128, 128), lambda i, bbase=bbase: (bbase + i, 0)
        ),
        out_shape=jax.ShapeDtypeStruct((B // 128, 128), jnp.float32),
        input_output_aliases=aliases,
    )(*args)


def kernel(X, user_table, movie_table, W1, b1, W2, b2):
    x32 = X.astype(jnp.int32)
    w1t = W1.T.astype(jnp.bfloat16)
    b1r = b1.reshape(H, 1)
    w2r = W2.reshape(1, H)
    b2r = b2.reshape(1, 1)

    acc = None
    start = 0
    for rows, bb in zip(CHUNKS, MLP_BB):
        u_c, m_c = _make_sc_gather(start, rows)(
            x32[start:start + rows, 0], x32[start:start + rows, 1],
            user_table, movie_table,
        )
        acc = _mlp_chunk(start, rows, bb, u_c, m_c, w1t, b1r, w2r, b2r, acc)
        start += rows
    return acc.reshape(B, 1)
